# baseline (speedup 1.0000x reference)
import functools

import jax
import jax.numpy as jnp
from jax import lax
from jax.experimental import pallas as pl
from jax.experimental.pallas import tpu as pltpu
from jax.experimental.pallas import tpu_sc as plsc

B, S, DIM = 4, 8192, 768
HID = DIM // 4
K = S // 2
L = 16


def _scores_body(x_ref, w1_ref, b1_ref, w2_ref, b2_ref, out_ref):
    sb = x_ref.shape[1]
    xb = x_ref[...].reshape(B * sb, DIM)
    h = lax.dot_general(xb, w1_ref[...], (((1,), (1,)), ((), ())),
                        preferred_element_type=jnp.float32)
    h = jnp.maximum(h + b1_ref[...], 0.0)
    sc = lax.dot_general(h.astype(jnp.bfloat16),
                         w2_ref[...].reshape(HID, 1).astype(jnp.bfloat16),
                         (((1,), (0,)), ((), ())),
                         preferred_element_type=jnp.float32)
    out_ref[...] = sc.reshape(B, sb) + b2_ref[0, 0]


def _scores_tc(x, W1, b1, W2, b2):
    SB = 2048
    return pl.pallas_call(
        _scores_body,
        grid=(S // SB,),
        in_specs=[
            pl.BlockSpec((B, SB, DIM), lambda j: (0, j, 0)),
            pl.BlockSpec((HID, DIM), lambda j: (0, 0)),
            pl.BlockSpec((1, HID), lambda j: (0, 0)),
            pl.BlockSpec((1, HID), lambda j: (0, 0)),
            pl.BlockSpec((1, 1), lambda j: (0, 0)),
        ],
        out_specs=pl.BlockSpec((B, SB), lambda j: (0, j)),
        out_shape=jax.ShapeDtypeStruct((B, S), jnp.float32),
    )(x, W1, b1.reshape(1, HID), W2.reshape(1, HID), b2.reshape(1, 1))


def _radix_pass(iota, hist, tot_v, base_v, dig_v, pos_v, src_k, dst_k,
                src_p, dst_p, shift, nbits, chunk):
    ndig = 1 << nbits
    dmask = ndig - 1
    lane_base = iota * chunk
    shift_v = jnp.full((L,), shift, jnp.int32)
    ones = jnp.ones((L,), jnp.int32)

    @plsc.parallel_loop(0, ndig, unroll=8)
    def _(h):
        hist[pl.ds(h * L, L)] = jnp.zeros((L,), jnp.int32)

    @plsc.parallel_loop(0, chunk, unroll=4)
    def _(v):
        key = plsc.load_gather(src_k, [lane_base + v])
        d = lax.shift_right_logical(key, shift_v) & dmask
        hidx = d * L + iota
        dig_v[pl.ds(v * L, L)] = hidx
        plsc.addupdate_scatter(hist, [hidx], ones)

    @plsc.parallel_loop(0, ndig, unroll=4)
    def _(h):
        sl = pl.ds(h * L, L)
        cnt = hist[sl]
        incl = plsc.cumsum(cnt)
        hist[sl] = incl - cnt
        tot_v[sl] = incl

    def scan2_body(j, carry):
        tv = plsc.load_gather(tot_v, [j * (L * L) + iota * L + (L - 1)])
        incl2 = plsc.cumsum(tv)
        base_v[pl.ds(j * L, L)] = incl2 - tv + carry
        return carry + incl2[L - 1]
    lax.fori_loop(0, ndig // L, scan2_body, jnp.int32(0))

    @plsc.parallel_loop(0, ndig, unroll=4)
    def _(h):
        sl = pl.ds(h * L, L)
        b = plsc.load_gather(base_v, [jnp.broadcast_to(h, (L,)).astype(jnp.int32)])
        hist[sl] = hist[sl] + b

    def pos_body(v, c):
        sl = pl.ds(v * L, L)
        hidx = dig_v[sl]
        pos = plsc.load_gather(hist, [hidx])
        plsc.store_scatter(hist, [hidx], pos + 1)
        pos_v[sl] = pos
        return c
    lax.fori_loop(0, chunk, pos_body, 0)

    @plsc.parallel_loop(0, chunk, unroll=4)
    def _(v):
        idx = lane_base + v
        pos = pos_v[pl.ds(v * L, L)]
        plsc.store_scatter(dst_k, [pos], plsc.load_gather(src_k, [idx]))
        if src_p is not None:
            plsc.store_scatter(dst_p, [pos], plsc.load_gather(src_p, [idx]))


def _router_body(scores_hbm, mask_hbm, rout_hbm,
                 s_v, ka, kb, ia, ib, hist, e_v, mask_v, rout_v,
                 dig_v, pos_v, tot_v, base_v):
    cid = lax.axis_index("c")
    sid = lax.axis_index("s")
    row = sid * 2 + cid

    @pl.when(row < B)
    def _():
        pltpu.sync_copy(scores_hbm.at[row], s_v)
        iota = lax.iota(jnp.int32, L)
        ones_i = jnp.ones((L,), jnp.int32)
        zeros_i = jnp.zeros((L,), jnp.int32)
        zeros_f = jnp.zeros((L,), jnp.float32)
        sh24 = jnp.full((L,), 24, jnp.int32)

        @plsc.parallel_loop(0, 256, unroll=8)
        def _(h):
            hist[pl.ds(h * L, L)] = jnp.zeros((L,), jnp.int32)

        @plsc.parallel_loop(0, S // L, unroll=4)
        def _(v):
            sl = pl.ds(v * L, L)
            i = lax.bitcast_convert_type(s_v[sl], jnp.int32)
            kd = jnp.where(i < 0, i, ~i & 0x7FFFFFFF)
            ka[sl] = kd
            d = lax.shift_right_logical(kd, sh24)
            plsc.addupdate_scatter(hist, [d * L + iota], ones_i)

        @plsc.parallel_loop(0, 256, unroll=4)
        def _(h):
            sl = pl.ds(h * L, L)
            tot_v[sl] = plsc.cumsum(hist[sl])

        def bstar_body(j, carry):
            cum, bst = carry
            tv = plsc.load_gather(tot_v, [j * (L * L) + iota * L + (L - 1)])
            incl2 = plsc.cumsum(tv) + cum
            cross = jnp.logical_and(incl2 >= K, incl2 - tv < K)
            bst = bst + jnp.sum(jnp.where(cross, j * L + iota, zeros_i))
            return (incl2[L - 1], bst)
        _, bstar = lax.fori_loop(0, 256 // L, bstar_body,
                                 (jnp.int32(0), jnp.int32(0)))
        bstar_v = jnp.broadcast_to(bstar, (L,))

        def class_body(v, cnt):
            sl = pl.ds(v * L, L)
            kd = ka[sl]
            m = lax.shift_right_logical(kd, sh24) <= bstar_v
            plsc.store_compressed(kb.at[pl.ds(cnt, L)], kd, mask=m)
            plsc.store_compressed(ib.at[pl.ds(cnt, L)], iota + v * L, mask=m)
            mask_v[sl] = zeros_i
            rout_v[sl] = zeros_f
            pc = plsc.all_reduce_population_count(m)
            return cnt + pc[0]
        nc = lax.fori_loop(0, S // L, class_body, jnp.int32(0))

        kb[pl.ds(nc, L)] = jnp.full((L,), -1, jnp.int32)
        ib[pl.ds(nc, L)] = zeros_i
        ncp = (nc + (L - 1)) & (-L)
        chunk = lax.shift_right_logical(ncp, 4)

        _radix_pass(iota, hist, tot_v, base_v, dig_v, pos_v, kb, ka, ib, ia,
                    0, 8, chunk)
        _radix_pass(iota, hist, tot_v, base_v, dig_v, pos_v, ka, kb, ia, ib,
                    8, 8, chunk)
        _radix_pass(iota, hist, tot_v, base_v, dig_v, pos_v, kb, ka, ib, ia,
                    16, 8, chunk)
        _radix_pass(iota, hist, tot_v, base_v, dig_v, pos_v, ka, kb, ia, ib,
                    24, 8, chunk)

        k0 = kb[pl.ds(0, L)]
        vmax = jnp.max(lax.bitcast_convert_type(
            jnp.where(k0 < 0, k0, ~k0 & 0x7FFFFFFF), jnp.float32))

        @plsc.parallel_loop(0, K // L, unroll=4,
                            carry=jnp.zeros((L,), jnp.float32))
        def zacc(r, acc):
            sl = pl.ds(r * L, L)
            kk = kb[sl]
            f = lax.bitcast_convert_type(
                jnp.where(kk < 0, kk, ~kk & 0x7FFFFFFF), jnp.float32)
            e = jnp.exp(f - vmax)
            e_v[sl] = e
            return acc + e
        zvec = jnp.broadcast_to(jnp.sum(zacc), (L,))
        zinv = jnp.ones((L,), jnp.float32) / zvec

        @plsc.parallel_loop(0, K // L, unroll=4)
        def _(r):
            plsc.store_scatter(mask_v, [ib[pl.ds(r * L, L)]], ones_i)

        def comp_body(v, cnt):
            sl = pl.ds(v * L, L)
            m = mask_v[sl] == 1
            plsc.store_compressed(ia.at[pl.ds(cnt, L)], iota + v * L, mask=m)
            pc = plsc.all_reduce_population_count(m)
            return cnt + pc[0]
        lax.fori_loop(0, S // L, comp_body, jnp.int32(0))

        @plsc.parallel_loop(0, K // L, unroll=4)
        def _(r):
            sl = pl.ds(r * L, L)
            plsc.store_scatter(rout_v, [ia[sl]], e_v[sl] * zinv)

        pltpu.sync_copy(mask_v, mask_hbm.at[row])
        pltpu.sync_copy(rout_v, rout_hbm.at[row])


def _router_sc(scores):
    mesh = plsc.VectorSubcoreMesh(core_axis_name="c", subcore_axis_name="s")
    fn = pl.kernel(
        _router_body,
        out_type=(jax.ShapeDtypeStruct((B, S), jnp.int32),
                  jax.ShapeDtypeStruct((B, S), jnp.float32)),
        mesh=mesh,
        compiler_params=pltpu.CompilerParams(needs_layout_passes=False),
        scratch_types=[
            pltpu.VMEM((S,), jnp.float32),
            pltpu.VMEM((S,), jnp.int32),
            pltpu.VMEM((S + L,), jnp.int32),
            pltpu.VMEM((S,), jnp.int32),
            pltpu.VMEM((S + L,), jnp.int32),
            pltpu.VMEM((256 * L,), jnp.int32),
            pltpu.VMEM((K,), jnp.float32),
            pltpu.VMEM((S,), jnp.int32),
            pltpu.VMEM((S,), jnp.float32),
            pltpu.VMEM((S,), jnp.int32),
            pltpu.VMEM((S,), jnp.int32),
            pltpu.VMEM((256 * L,), jnp.int32),
            pltpu.VMEM((256,), jnp.int32),
        ],
    )
    return fn(scores)


def kernel(x, W1, b1, W2, b2):
    scores = _scores_tc(x, W1, b1, W2, b2)
    mask_i, routing = _router_sc(scores)
    return mask_i.astype(bool), routing


# speedup vs baseline: 1.0236x; 1.0236x over previous
import jax
import jax.numpy as jnp
from jax import lax
from jax.experimental import pallas as pl
from jax.experimental.pallas import tpu as pltpu
from jax.experimental.pallas import tpu_sc as plsc

B, S, DIM = 4, 8192, 768
HID = DIM // 4
K = S // 2
L = 16


def _scores_body(x_ref, w1_ref, b1_ref, w2_ref, b2_ref, out_ref):
    sb = x_ref.shape[1]
    xb = x_ref[...].reshape(B * sb, DIM)
    h = lax.dot_general(xb, w1_ref[...], (((1,), (1,)), ((), ())),
                        preferred_element_type=jnp.float32)
    h = jnp.maximum(h + b1_ref[...], 0.0)
    sc = lax.dot_general(h.astype(jnp.bfloat16),
                         w2_ref[...].reshape(HID, 1).astype(jnp.bfloat16),
                         (((1,), (0,)), ((), ())),
                         preferred_element_type=jnp.float32)
    out_ref[...] = sc.reshape(B, sb) + b2_ref[0, 0]


def _scores_tc(x, W1, b1, W2, b2):
    SB = 1024
    return pl.pallas_call(
        _scores_body,
        grid=(S // SB,),
        in_specs=[
            pl.BlockSpec((B, SB, DIM), lambda j: (0, j, 0)),
            pl.BlockSpec((HID, DIM), lambda j: (0, 0)),
            pl.BlockSpec((1, HID), lambda j: (0, 0)),
            pl.BlockSpec((1, HID), lambda j: (0, 0)),
            pl.BlockSpec((1, 1), lambda j: (0, 0)),
        ],
        out_specs=pl.BlockSpec((B, SB), lambda j: (0, j)),
        out_shape=jax.ShapeDtypeStruct((B, S), jnp.float32),
    )(x, W1, b1.reshape(1, HID), W2.reshape(1, HID), b2.reshape(1, 1))


def _radix_pass(iota, hist, tot_v, base_v, dig_v, pos_v, src_k, dst_k,
                src_p, dst_p, shift, nbits, chunk):
    ndig = 1 << nbits
    dmask = ndig - 1
    lane_base = iota * chunk
    shift_v = jnp.full((L,), shift, jnp.int32)
    ones = jnp.ones((L,), jnp.int32)

    @plsc.parallel_loop(0, ndig, unroll=8)
    def _(h):
        hist[pl.ds(h * L, L)] = jnp.zeros((L,), jnp.int32)

    @plsc.parallel_loop(0, chunk, unroll=4)
    def _(v):
        key = plsc.load_gather(src_k, [lane_base + v])
        d = lax.shift_right_logical(key, shift_v) & dmask
        hidx = d * L + iota
        dig_v[pl.ds(v * L, L)] = hidx
        plsc.addupdate_scatter(hist, [hidx], ones)

    @plsc.parallel_loop(0, ndig, unroll=4)
    def _(h):
        sl = pl.ds(h * L, L)
        cnt = hist[sl]
        incl = plsc.cumsum(cnt)
        hist[sl] = incl - cnt
        tot_v[sl] = incl

    def scan2_body(j, carry):
        tv = plsc.load_gather(tot_v, [j * (L * L) + iota * L + (L - 1)])
        incl2 = plsc.cumsum(tv)
        base_v[pl.ds(j * L, L)] = incl2 - tv + carry
        return carry + incl2[L - 1]
    lax.fori_loop(0, ndig // L, scan2_body, jnp.int32(0))

    @plsc.parallel_loop(0, ndig, unroll=4)
    def _(h):
        sl = pl.ds(h * L, L)
        b = plsc.load_gather(base_v, [jnp.broadcast_to(h, (L,)).astype(jnp.int32)])
        hist[sl] = hist[sl] + b

    def pos_body(v, c):
        sl = pl.ds(v * L, L)
        hidx = dig_v[sl]
        pos = plsc.load_gather(hist, [hidx])
        plsc.store_scatter(hist, [hidx], pos + 1)
        pos_v[sl] = pos
        return c
    lax.fori_loop(0, chunk, pos_body, 0)

    @plsc.parallel_loop(0, chunk, unroll=4)
    def _(v):
        idx = lane_base + v
        pos = pos_v[pl.ds(v * L, L)]
        plsc.store_scatter(dst_k, [pos], plsc.load_gather(src_k, [idx]))
        if src_p is not None:
            plsc.store_scatter(dst_p, [pos], plsc.load_gather(src_p, [idx]))


def _router_body(scores_hbm, mask_hbm, rout_hbm,
                 s_v, ka, kb, ia, ib, hist, e_v, mask_v, rout_v,
                 dig_v, pos_v, tot_v, base_v):
    cid = lax.axis_index("c")
    sid = lax.axis_index("s")
    row = sid * 2 + cid

    @pl.when(row < B)
    def _():
        pltpu.sync_copy(scores_hbm.at[row], s_v)
        iota = lax.iota(jnp.int32, L)
        ones_i = jnp.ones((L,), jnp.int32)
        zeros_i = jnp.zeros((L,), jnp.int32)
        zeros_f = jnp.zeros((L,), jnp.float32)
        sh24 = jnp.full((L,), 24, jnp.int32)

        @plsc.parallel_loop(0, 256, unroll=8)
        def _(h):
            hist[pl.ds(h * L, L)] = jnp.zeros((L,), jnp.int32)

        @plsc.parallel_loop(0, S // L, unroll=4)
        def _(v):
            sl = pl.ds(v * L, L)
            i = lax.bitcast_convert_type(s_v[sl], jnp.int32)
            kd = jnp.where(i < 0, i, ~i & 0x7FFFFFFF)
            ka[sl] = kd
            d = lax.shift_right_logical(kd, sh24)
            plsc.addupdate_scatter(hist, [d * L + iota], ones_i)

        @plsc.parallel_loop(0, 256, unroll=4)
        def _(h):
            sl = pl.ds(h * L, L)
            tot_v[sl] = plsc.cumsum(hist[sl])

        def bstar_body(j, carry):
            cum, bst = carry
            tv = plsc.load_gather(tot_v, [j * (L * L) + iota * L + (L - 1)])
            incl2 = plsc.cumsum(tv) + cum
            cross = jnp.logical_and(incl2 >= K, incl2 - tv < K)
            bst = bst + jnp.sum(jnp.where(cross, j * L + iota, zeros_i))
            return (incl2[L - 1], bst)
        _, bstar = lax.fori_loop(0, 256 // L, bstar_body,
                                 (jnp.int32(0), jnp.int32(0)))
        bstar_v = jnp.broadcast_to(bstar, (L,))

        def class_body(v, cnt):
            sl = pl.ds(v * L, L)
            kd = ka[sl]
            m = lax.shift_right_logical(kd, sh24) <= bstar_v
            plsc.store_compressed(kb.at[pl.ds(cnt, L)], kd, mask=m)
            plsc.store_compressed(ib.at[pl.ds(cnt, L)], iota + v * L, mask=m)
            mask_v[sl] = zeros_i
            rout_v[sl] = zeros_f
            pc = plsc.all_reduce_population_count(m)
            return cnt + pc[0]
        nc = lax.fori_loop(0, S // L, class_body, jnp.int32(0))

        kb[pl.ds(nc, L)] = jnp.full((L,), -1, jnp.int32)
        ib[pl.ds(nc, L)] = zeros_i
        ncp = (nc + (L - 1)) & (-L)
        chunk = lax.shift_right_logical(ncp, 4)

        _radix_pass(iota, hist, tot_v, base_v, dig_v, pos_v, kb, ka, ib, ia,
                    0, 8, chunk)
        _radix_pass(iota, hist, tot_v, base_v, dig_v, pos_v, ka, kb, ia, ib,
                    8, 8, chunk)
        _radix_pass(iota, hist, tot_v, base_v, dig_v, pos_v, kb, ka, ib, ia,
                    16, 8, chunk)
        _radix_pass(iota, hist, tot_v, base_v, dig_v, pos_v, ka, kb, ia, ib,
                    24, 8, chunk)

        k0 = kb[pl.ds(0, L)]
        vmax = jnp.max(lax.bitcast_convert_type(
            jnp.where(k0 < 0, k0, ~k0 & 0x7FFFFFFF), jnp.float32))

        @plsc.parallel_loop(0, K // L, unroll=4,
                            carry=jnp.zeros((L,), jnp.float32))
        def zacc(r, acc):
            sl = pl.ds(r * L, L)
            kk = kb[sl]
            f = lax.bitcast_convert_type(
                jnp.where(kk < 0, kk, ~kk & 0x7FFFFFFF), jnp.float32)
            e = jnp.exp(f - vmax)
            e_v[sl] = e
            return acc + e
        zvec = jnp.broadcast_to(jnp.sum(zacc), (L,))
        zinv = jnp.ones((L,), jnp.float32) / zvec

        @plsc.parallel_loop(0, K // L, unroll=4)
        def _(r):
            plsc.store_scatter(mask_v, [ib[pl.ds(r * L, L)]], ones_i)

        def comp_body(v, cnt):
            sl = pl.ds(v * L, L)
            m = mask_v[sl] == 1
            plsc.store_compressed(ia.at[pl.ds(cnt, L)], iota + v * L, mask=m)
            pc = plsc.all_reduce_population_count(m)
            return cnt + pc[0]
        lax.fori_loop(0, S // L, comp_body, jnp.int32(0))

        @plsc.parallel_loop(0, K // L, unroll=4)
        def _(r):
            sl = pl.ds(r * L, L)
            plsc.store_scatter(rout_v, [ia[sl]], e_v[sl] * zinv)

        pltpu.sync_copy(mask_v, mask_hbm.at[row])
        pltpu.sync_copy(rout_v, rout_hbm.at[row])


def _router_sc(scores):
    mesh = plsc.VectorSubcoreMesh(core_axis_name="c", subcore_axis_name="s")
    fn = pl.kernel(
        _router_body,
        out_type=(jax.ShapeDtypeStruct((B, S), jnp.int32),
                  jax.ShapeDtypeStruct((B, S), jnp.float32)),
        mesh=mesh,
        compiler_params=pltpu.CompilerParams(needs_layout_passes=False),
        scratch_types=[
            pltpu.VMEM((S,), jnp.float32),
            pltpu.VMEM((S,), jnp.int32),
            pltpu.VMEM((S + L,), jnp.int32),
            pltpu.VMEM((S,), jnp.int32),
            pltpu.VMEM((S + L,), jnp.int32),
            pltpu.VMEM((256 * L,), jnp.int32),
            pltpu.VMEM((K,), jnp.float32),
            pltpu.VMEM((S,), jnp.int32),
            pltpu.VMEM((S,), jnp.float32),
            pltpu.VMEM((S,), jnp.int32),
            pltpu.VMEM((S,), jnp.int32),
            pltpu.VMEM((256 * L,), jnp.int32),
            pltpu.VMEM((256,), jnp.int32),
        ],
    )
    return fn(scores)


def kernel(x, W1, b1, W2, b2):
    scores = _scores_tc(x, W1, b1, W2, b2)
    mask_i, routing = _router_sc(scores)
    return mask_i.astype(bool), routing
